# Initial kernel scaffold; baseline (speedup 1.0000x reference)
import jax
import jax.numpy as jnp
from jax.experimental import pallas as pl


def kernel(x, edge_index, W1, b1, W2, b2, Wo, bo):
    raise NotImplementedError("write your pallas kernel here")


# speedup vs baseline: 26.5827x; 26.5827x over previous
import functools

import jax
import jax.numpy as jnp
from jax import lax
from jax.experimental import pallas as pl
from jax.experimental.pallas import tpu as pltpu
from jax.experimental.pallas import tpu_sc as plsc

N = 10000
E = 320000
D = 128
H = 64

NC = 2
NS = 16
NW = NC * NS

EPB = 128
NB = -(-E // (NW * EPB))
EPAD = NW * NB * EPB
NPAD = 10240
RPS = NPAD // NS
DEGW = 16

BN = 640

_MESH = dict(core_axis_name="c", subcore_axis_name="s")
_SC_PARAMS = pltpu.CompilerParams(use_tc_tiling_on_sc=False)


@functools.partial(
    pl.kernel,
    mesh=plsc.VectorSubcoreMesh(**_MESH),
    out_type=jax.ShapeDtypeStruct((NC, NPAD, DEGW), jnp.float32),
    scratch_types=[
        pltpu.VMEM((NB, EPB), jnp.int32),
        pltpu.VMEM((EPB, DEGW), jnp.float32),
        pltpu.VMEM_SHARED((NPAD, DEGW), jnp.float32),
    ],
    compiler_params=_SC_PARAMS,
)
def _sc_degree(dst_hbm, ones_hbm, zero_hbm, out_hbm, idx_v, ones_v, acc_sh):
    cid = lax.axis_index("c")
    sid = lax.axis_index("s")
    wid = sid * NC + cid
    pltpu.sync_copy(zero_hbm.at[pl.ds(sid * RPS, RPS)],
                    acc_sh.at[pl.ds(sid * RPS, RPS)])
    pltpu.sync_copy(dst_hbm.at[wid], idx_v)
    pltpu.sync_copy(ones_hbm, ones_v)
    plsc.subcore_barrier()

    def body(j, carry):
        pltpu.sync_copy(ones_v, acc_sh.at[idx_v.at[j]], add=True)
        return carry

    lax.fori_loop(0, NB, body, 0)
    plsc.subcore_barrier()
    pltpu.sync_copy(acc_sh.at[pl.ds(sid * RPS, RPS)],
                    out_hbm.at[cid, pl.ds(sid * RPS, RPS)])


@functools.partial(
    pl.kernel,
    mesh=plsc.VectorSubcoreMesh(**_MESH),
    out_type=jax.ShapeDtypeStruct((NC, NPAD, H), jnp.float32),
    scratch_types=[
        pltpu.VMEM((NB, EPB), jnp.int32),
        pltpu.VMEM((NB, EPB), jnp.int32),
        pltpu.VMEM((EPB, H), jnp.float32),
        pltpu.VMEM_SHARED((NPAD, H), jnp.float32),
        pltpu.SemaphoreType.DMA,
    ],
    compiler_params=_SC_PARAMS,
)
def _sc_edge_sum(g_hbm, src_hbm, dst_hbm, zero_hbm, out_hbm,
                 src_v, dst_v, rows_v, acc_sh, sem):
    cid = lax.axis_index("c")
    sid = lax.axis_index("s")
    wid = sid * NC + cid
    pltpu.sync_copy(zero_hbm.at[pl.ds(sid * RPS, RPS)],
                    acc_sh.at[pl.ds(sid * RPS, RPS)])
    pltpu.sync_copy(src_hbm.at[wid], src_v)
    pltpu.sync_copy(dst_hbm.at[wid], dst_v)
    plsc.subcore_barrier()

    def body(j, carry):
        pltpu.async_copy(g_hbm.at[src_v.at[j]], rows_v, sem).wait()
        pltpu.sync_copy(rows_v, acc_sh.at[dst_v.at[j]], add=True)
        return carry

    lax.fori_loop(0, NB, body, 0)
    plsc.subcore_barrier()
    pltpu.sync_copy(acc_sh.at[pl.ds(sid * RPS, RPS)],
                    out_hbm.at[cid, pl.ds(sid * RPS, RPS)])


def _tc_stage1(x_ref, w_ref, dp_ref, g_ref, dis_ref):
    i = pl.program_id(0)
    dp = dp_ref[...]
    deg = dp[0, :, 0:1] + dp[1, :, 0:1] + 1.0
    row = lax.broadcasted_iota(jnp.int32, (BN, 1), 0) + i * BN
    dis = jnp.where(row < N, lax.rsqrt(deg), 0.0)
    dis_ref[...] = dis
    g_ref[...] = jnp.dot(x_ref[...], w_ref[...],
                         preferred_element_type=jnp.float32) * dis


def _tc_stage2(p_ref, g_ref, dis_ref, b_ref, w_ref, out_ref):
    p = p_ref[...]
    dis = dis_ref[...]
    s = p[0] + p[1] + g_ref[...]
    h = jnp.maximum(s * dis + b_ref[...], 0.0)
    out_ref[...] = jnp.dot(h, w_ref[...],
                           preferred_element_type=jnp.float32) * dis


def _tc_stage3(p_ref, g_ref, dis_ref, b_ref, wo_ref, bo_ref, out_ref):
    p = p_ref[...]
    s = p[0] + p[1] + g_ref[...]
    h = jnp.maximum(s * dis_ref[...] + b_ref[...], 0.0)
    out_ref[...] = jnp.dot(h, wo_ref[...],
                           preferred_element_type=jnp.float32) + bo_ref[...]


def _rows(bn, cols):
    return pl.BlockSpec((bn, cols), lambda i: (i, 0))


def _full(shape):
    return pl.BlockSpec(shape, lambda i: tuple(0 for _ in shape))


def _partials(cols):
    return pl.BlockSpec((NC, BN, cols), lambda i: (0, i, 0))


_GRID = NPAD // BN

_stage1 = pl.pallas_call(
    _tc_stage1,
    grid=(_GRID,),
    in_specs=[_rows(BN, D), _full((D, H)), _partials(DEGW)],
    out_specs=[_rows(BN, H), _rows(BN, 1)],
    out_shape=[jax.ShapeDtypeStruct((NPAD, H), jnp.float32),
               jax.ShapeDtypeStruct((NPAD, 1), jnp.float32)],
)

_stage2 = pl.pallas_call(
    _tc_stage2,
    grid=(_GRID,),
    in_specs=[_partials(H), _rows(BN, H), _rows(BN, 1), _full((1, H)),
              _full((H, H))],
    out_specs=_rows(BN, H),
    out_shape=jax.ShapeDtypeStruct((NPAD, H), jnp.float32),
)

_stage3 = pl.pallas_call(
    _tc_stage3,
    grid=(_GRID,),
    in_specs=[_partials(H), _rows(BN, H), _rows(BN, 1), _full((1, H)),
              _full((H, 1)), _full((1, 1))],
    out_specs=_rows(BN, 1),
    out_shape=jax.ShapeDtypeStruct((NPAD, 1), jnp.float32),
)


def kernel(x, edge_index, W1, b1, W2, b2, Wo, bo):
    f32 = jnp.float32
    src = edge_index[0]
    dst = edge_index[1]
    pad = EPAD - E
    pad_idx = (N + (jnp.arange(pad, dtype=jnp.int32) % (NPAD - N)))
    src3 = jnp.concatenate([src, pad_idx]).reshape(NW, NB, EPB)
    dst3 = jnp.concatenate([dst, pad_idx]).reshape(NW, NB, EPB)

    zeros_h = jnp.zeros((NPAD, H), f32)
    zeros_deg = jnp.zeros((NPAD, DEGW), f32)
    ones_deg = jnp.ones((EPB, DEGW), f32)
    x_pad = jnp.pad(x, ((0, NPAD - N), (0, 0)))

    deg_part = _sc_degree(dst3, ones_deg, zeros_deg)

    g1, dis = _stage1(x_pad, W1, deg_part)
    s1 = _sc_edge_sum(g1, src3, dst3, zeros_h)
    g2 = _stage2(s1, g1, dis, b1.reshape(1, H), W2)
    s2 = _sc_edge_sum(g2, src3, dst3, zeros_h)
    y = _stage3(s2, g2, dis, b2.reshape(1, H), Wo, bo.reshape(1, 1))
    return y[:N, 0]


# speedup vs baseline: 35.1260x; 1.3214x over previous
import functools

import jax
import jax.numpy as jnp
from jax import lax
from jax.experimental import pallas as pl
from jax.experimental.pallas import tpu as pltpu
from jax.experimental.pallas import tpu_sc as plsc

N = 10000
E = 320000
D = 128
H = 64

NC = 2
NS = 16
NW = NC * NS

EPB = 128
NB = 2 * (-(-E // (NW * EPB * 2)))
EPAD = NW * NB * EPB
NPAD = 10240
RPS = NPAD // NS
DEGW = 1

BN = 640

_MESH = dict(core_axis_name="c", subcore_axis_name="s")
_SC_PARAMS = pltpu.CompilerParams(use_tc_tiling_on_sc=False)


@functools.partial(
    pl.kernel,
    mesh=plsc.VectorSubcoreMesh(**_MESH),
    out_type=jax.ShapeDtypeStruct((NC, NPAD, DEGW), jnp.float32),
    scratch_types=[
        pltpu.VMEM((NB, EPB), jnp.int32),
        pltpu.VMEM((EPB, DEGW), jnp.float32),
        pltpu.VMEM_SHARED((NPAD, DEGW), jnp.float32),
    ],
    compiler_params=_SC_PARAMS,
)
def _sc_degree(dst_hbm, ones_hbm, zero_hbm, out_hbm, idx_v, ones_v, acc_sh):
    cid = lax.axis_index("c")
    sid = lax.axis_index("s")
    wid = sid * NC + cid
    pltpu.sync_copy(zero_hbm.at[pl.ds(sid * RPS, RPS)],
                    acc_sh.at[pl.ds(sid * RPS, RPS)])
    pltpu.sync_copy(dst_hbm.at[wid], idx_v)
    pltpu.sync_copy(ones_hbm, ones_v)
    plsc.subcore_barrier()

    def body(j, carry):
        pltpu.sync_copy(ones_v, acc_sh.at[idx_v.at[j]], add=True)
        return carry

    lax.fori_loop(0, NB, body, 0)
    plsc.subcore_barrier()
    pltpu.sync_copy(acc_sh.at[pl.ds(sid * RPS, RPS)],
                    out_hbm.at[cid, pl.ds(sid * RPS, RPS)])


@functools.partial(
    pl.kernel,
    mesh=plsc.VectorSubcoreMesh(**_MESH),
    out_type=jax.ShapeDtypeStruct((NC, NPAD, H), jnp.float32),
    scratch_types=[
        pltpu.VMEM((NB, EPB), jnp.int32),
        pltpu.VMEM((NB, EPB), jnp.int32),
        pltpu.VMEM((EPB, H), jnp.float32),
        pltpu.VMEM((EPB, H), jnp.float32),
        pltpu.VMEM_SHARED((NPAD, H), jnp.float32),
        pltpu.SemaphoreType.DMA,
        pltpu.SemaphoreType.DMA,
    ],
    compiler_params=_SC_PARAMS,
)
def _sc_edge_sum(g_hbm, src_hbm, dst_hbm, zero_hbm, out_hbm,
                 src_v, dst_v, rows0, rows1, acc_sh, sem0, sem1):
    cid = lax.axis_index("c")
    sid = lax.axis_index("s")
    wid = sid * NC + cid
    pltpu.sync_copy(zero_hbm.at[pl.ds(sid * RPS, RPS)],
                    acc_sh.at[pl.ds(sid * RPS, RPS)])
    pltpu.sync_copy(src_hbm.at[wid], src_v)
    pltpu.sync_copy(dst_hbm.at[wid], dst_v)
    plsc.subcore_barrier()

    pltpu.async_copy(g_hbm.at[src_v.at[0]], rows0, sem0)

    def body(t, carry):
        j0 = 2 * t
        j1 = j0 + 1
        j2 = j0 + 2
        pltpu.async_copy(g_hbm.at[src_v.at[j1]], rows1, sem1)
        pltpu.make_async_copy(g_hbm.at[src_v.at[j0]], rows0, sem0).wait()
        pltpu.sync_copy(rows0, acc_sh.at[dst_v.at[j0]], add=True)

        @pl.when(j2 < NB)
        def _():
            pltpu.async_copy(g_hbm.at[src_v.at[j2]], rows0, sem0)

        pltpu.make_async_copy(g_hbm.at[src_v.at[j1]], rows1, sem1).wait()
        pltpu.sync_copy(rows1, acc_sh.at[dst_v.at[j1]], add=True)
        return carry

    lax.fori_loop(0, NB // 2, body, 0)
    plsc.subcore_barrier()
    pltpu.sync_copy(acc_sh.at[pl.ds(sid * RPS, RPS)],
                    out_hbm.at[cid, pl.ds(sid * RPS, RPS)])


def _tc_stage1(x_ref, w_ref, dp_ref, g_ref, dis_ref):
    i = pl.program_id(0)
    dp = dp_ref[...]
    deg = dp[0, :, 0:1] + dp[1, :, 0:1] + 1.0
    row = lax.broadcasted_iota(jnp.int32, (BN, 1), 0) + i * BN
    dis = jnp.where(row < N, lax.rsqrt(deg), 0.0)
    dis_ref[...] = dis
    g_ref[...] = jnp.dot(x_ref[...], w_ref[...],
                         preferred_element_type=jnp.float32) * dis


def _tc_stage2(p_ref, g_ref, dis_ref, b_ref, w_ref, out_ref):
    p = p_ref[...]
    dis = dis_ref[...]
    s = p[0] + p[1] + g_ref[...]
    h = jnp.maximum(s * dis + b_ref[...], 0.0)
    out_ref[...] = jnp.dot(h, w_ref[...],
                           preferred_element_type=jnp.float32) * dis


def _tc_stage3(p_ref, g_ref, dis_ref, b_ref, wo_ref, bo_ref, out_ref):
    p = p_ref[...]
    s = p[0] + p[1] + g_ref[...]
    h = jnp.maximum(s * dis_ref[...] + b_ref[...], 0.0)
    out_ref[...] = jnp.dot(h, wo_ref[...],
                           preferred_element_type=jnp.float32) + bo_ref[...]


def _rows(bn, cols):
    return pl.BlockSpec((bn, cols), lambda i: (i, 0))


def _full(shape):
    return pl.BlockSpec(shape, lambda i: tuple(0 for _ in shape))


def _partials(cols):
    return pl.BlockSpec((NC, BN, cols), lambda i: (0, i, 0))


_GRID = NPAD // BN

_stage1 = pl.pallas_call(
    _tc_stage1,
    grid=(_GRID,),
    in_specs=[_rows(BN, D), _full((D, H)), _partials(DEGW)],
    out_specs=[_rows(BN, H), _rows(BN, 1)],
    out_shape=[jax.ShapeDtypeStruct((NPAD, H), jnp.float32),
               jax.ShapeDtypeStruct((NPAD, 1), jnp.float32)],
)

_stage2 = pl.pallas_call(
    _tc_stage2,
    grid=(_GRID,),
    in_specs=[_partials(H), _rows(BN, H), _rows(BN, 1), _full((1, H)),
              _full((H, H))],
    out_specs=_rows(BN, H),
    out_shape=jax.ShapeDtypeStruct((NPAD, H), jnp.float32),
)

_stage3 = pl.pallas_call(
    _tc_stage3,
    grid=(_GRID,),
    in_specs=[_partials(H), _rows(BN, H), _rows(BN, 1), _full((1, H)),
              _full((H, 1)), _full((1, 1))],
    out_specs=_rows(BN, 1),
    out_shape=jax.ShapeDtypeStruct((NPAD, 1), jnp.float32),
)


def kernel(x, edge_index, W1, b1, W2, b2, Wo, bo):
    f32 = jnp.float32
    src = edge_index[0]
    dst = edge_index[1]
    pad = EPAD - E
    pad_idx = (N + (jnp.arange(pad, dtype=jnp.int32) % (NPAD - N)))
    src3 = jnp.concatenate([src, pad_idx]).reshape(NW, NB, EPB)
    dst3 = jnp.concatenate([dst, pad_idx]).reshape(NW, NB, EPB)

    zeros_h = jnp.zeros((NPAD, H), f32)
    zeros_deg = jnp.zeros((NPAD, DEGW), f32)
    ones_deg = jnp.ones((EPB, DEGW), f32)
    x_pad = jnp.pad(x, ((0, NPAD - N), (0, 0)))

    deg_part = _sc_degree(dst3, ones_deg, zeros_deg)

    g1, dis = _stage1(x_pad, W1, deg_part)
    s1 = _sc_edge_sum(g1, src3, dst3, zeros_h)
    g2 = _stage2(s1, g1, dis, b1.reshape(1, H), W2)
    s2 = _sc_edge_sum(g2, src3, dst3, zeros_h)
    y = _stage3(s2, g2, dis, b2.reshape(1, H), Wo, bo.reshape(1, 1))
    return y[:N, 0]


# speedup vs baseline: 38.2031x; 1.0876x over previous
import functools

import jax
import jax.numpy as jnp
from jax import lax
from jax.experimental import pallas as pl
from jax.experimental.pallas import tpu as pltpu
from jax.experimental.pallas import tpu_sc as plsc

N = 10000
E = 320000
D = 128
H = 64

NC = 2
NS = 16
NW = NC * NS

EPB = 512
NB = 20
EPAD = NW * NB * EPB
NPAD = 10240
RPS = NPAD // NS
DEGW = 1

BN = 640

_MESH = dict(core_axis_name="c", subcore_axis_name="s")
_SC_PARAMS = pltpu.CompilerParams(use_tc_tiling_on_sc=False)


@functools.partial(
    pl.kernel,
    mesh=plsc.VectorSubcoreMesh(**_MESH),
    out_type=jax.ShapeDtypeStruct((NC, NPAD, DEGW), jnp.float32),
    scratch_types=[
        pltpu.VMEM((NB, EPB), jnp.int32),
        pltpu.VMEM((EPB, DEGW), jnp.float32),
        pltpu.VMEM_SHARED((NPAD, DEGW), jnp.float32),
    ],
    compiler_params=_SC_PARAMS,
)
def _sc_degree(dst_hbm, ones_hbm, zero_hbm, out_hbm, idx_v, ones_v, acc_sh):
    cid = lax.axis_index("c")
    sid = lax.axis_index("s")
    wid = sid * NC + cid
    pltpu.sync_copy(zero_hbm.at[pl.ds(sid * RPS, RPS)],
                    acc_sh.at[pl.ds(sid * RPS, RPS)])
    pltpu.sync_copy(dst_hbm.at[wid], idx_v)
    pltpu.sync_copy(ones_hbm, ones_v)
    plsc.subcore_barrier()

    def body(j, carry):
        pltpu.sync_copy(ones_v, acc_sh.at[idx_v.at[j]], add=True)
        return carry

    lax.fori_loop(0, NB, body, 0)
    plsc.subcore_barrier()
    pltpu.sync_copy(acc_sh.at[pl.ds(sid * RPS, RPS)],
                    out_hbm.at[cid, pl.ds(sid * RPS, RPS)])


@functools.partial(
    pl.kernel,
    mesh=plsc.VectorSubcoreMesh(**_MESH),
    out_type=jax.ShapeDtypeStruct((NC, NPAD, H), jnp.float32),
    scratch_types=[
        pltpu.VMEM((NB, EPB), jnp.int32),
        pltpu.VMEM((NB, EPB), jnp.int32),
        pltpu.VMEM((EPB, H), jnp.float32),
        pltpu.VMEM((EPB, H), jnp.float32),
        pltpu.VMEM_SHARED((NPAD, H), jnp.float32),
        pltpu.SemaphoreType.DMA,
        pltpu.SemaphoreType.DMA,
    ],
    compiler_params=_SC_PARAMS,
)
def _sc_edge_sum(g_hbm, src_hbm, dst_hbm, zero_hbm, out_hbm,
                 src_v, dst_v, rows0, rows1, acc_sh, sem0, sem1):
    cid = lax.axis_index("c")
    sid = lax.axis_index("s")
    wid = sid * NC + cid
    pltpu.sync_copy(zero_hbm.at[pl.ds(sid * RPS, RPS)],
                    acc_sh.at[pl.ds(sid * RPS, RPS)])
    pltpu.sync_copy(src_hbm.at[wid], src_v)
    pltpu.sync_copy(dst_hbm.at[wid], dst_v)
    plsc.subcore_barrier()

    pltpu.async_copy(g_hbm.at[src_v.at[0]], rows0, sem0)

    def body(t, carry):
        j0 = 2 * t
        j1 = j0 + 1
        j2 = j0 + 2
        pltpu.async_copy(g_hbm.at[src_v.at[j1]], rows1, sem1)
        pltpu.make_async_copy(g_hbm.at[src_v.at[j0]], rows0, sem0).wait()
        pltpu.sync_copy(rows0, acc_sh.at[dst_v.at[j0]], add=True)

        @pl.when(j2 < NB)
        def _():
            pltpu.async_copy(g_hbm.at[src_v.at[j2]], rows0, sem0)

        pltpu.make_async_copy(g_hbm.at[src_v.at[j1]], rows1, sem1).wait()
        pltpu.sync_copy(rows1, acc_sh.at[dst_v.at[j1]], add=True)
        return carry

    lax.fori_loop(0, NB // 2, body, 0)
    plsc.subcore_barrier()
    pltpu.sync_copy(acc_sh.at[pl.ds(sid * RPS, RPS)],
                    out_hbm.at[cid, pl.ds(sid * RPS, RPS)])


def _tc_stage1(x_ref, w_ref, dp_ref, g_ref, dis_ref):
    i = pl.program_id(0)
    dp = dp_ref[...]
    deg = dp[0, :, 0:1] + dp[1, :, 0:1] + 1.0
    row = lax.broadcasted_iota(jnp.int32, (BN, 1), 0) + i * BN
    dis = jnp.where(row < N, lax.rsqrt(deg), 0.0)
    dis_ref[...] = dis
    g_ref[...] = jnp.dot(x_ref[...], w_ref[...],
                         preferred_element_type=jnp.float32) * dis


def _tc_stage2(p_ref, g_ref, dis_ref, b_ref, w_ref, out_ref):
    p = p_ref[...]
    dis = dis_ref[...]
    s = p[0] + p[1] + g_ref[...]
    h = jnp.maximum(s * dis + b_ref[...], 0.0)
    out_ref[...] = jnp.dot(h, w_ref[...],
                           preferred_element_type=jnp.float32) * dis


def _tc_stage3(p_ref, g_ref, dis_ref, b_ref, wo_ref, bo_ref, out_ref):
    p = p_ref[...]
    s = p[0] + p[1] + g_ref[...]
    h = jnp.maximum(s * dis_ref[...] + b_ref[...], 0.0)
    out_ref[...] = jnp.dot(h, wo_ref[...],
                           preferred_element_type=jnp.float32) + bo_ref[...]


def _rows(bn, cols):
    return pl.BlockSpec((bn, cols), lambda i: (i, 0))


def _full(shape):
    return pl.BlockSpec(shape, lambda i: tuple(0 for _ in shape))


def _partials(cols):
    return pl.BlockSpec((NC, BN, cols), lambda i: (0, i, 0))


_GRID = NPAD // BN

_stage1 = pl.pallas_call(
    _tc_stage1,
    grid=(_GRID,),
    in_specs=[_rows(BN, D), _full((D, H)), _partials(DEGW)],
    out_specs=[_rows(BN, H), _rows(BN, 1)],
    out_shape=[jax.ShapeDtypeStruct((NPAD, H), jnp.float32),
               jax.ShapeDtypeStruct((NPAD, 1), jnp.float32)],
)

_stage2 = pl.pallas_call(
    _tc_stage2,
    grid=(_GRID,),
    in_specs=[_partials(H), _rows(BN, H), _rows(BN, 1), _full((1, H)),
              _full((H, H))],
    out_specs=_rows(BN, H),
    out_shape=jax.ShapeDtypeStruct((NPAD, H), jnp.float32),
)

_stage3 = pl.pallas_call(
    _tc_stage3,
    grid=(_GRID,),
    in_specs=[_partials(H), _rows(BN, H), _rows(BN, 1), _full((1, H)),
              _full((H, 1)), _full((1, 1))],
    out_specs=_rows(BN, 1),
    out_shape=jax.ShapeDtypeStruct((NPAD, 1), jnp.float32),
)


def kernel(x, edge_index, W1, b1, W2, b2, Wo, bo):
    f32 = jnp.float32
    src = edge_index[0]
    dst = edge_index[1]
    pad = EPAD - E
    pad_idx = (N + (jnp.arange(pad, dtype=jnp.int32) % (NPAD - N)))
    src3 = jnp.concatenate([src, pad_idx]).reshape(NW, NB, EPB)
    dst3 = jnp.concatenate([dst, pad_idx]).reshape(NW, NB, EPB)

    zeros_h = jnp.zeros((NPAD, H), f32)
    zeros_deg = jnp.zeros((NPAD, DEGW), f32)
    ones_deg = jnp.ones((EPB, DEGW), f32)
    x_pad = jnp.pad(x, ((0, NPAD - N), (0, 0)))

    deg_part = _sc_degree(dst3, ones_deg, zeros_deg)

    g1, dis = _stage1(x_pad, W1, deg_part)
    s1 = _sc_edge_sum(g1, src3, dst3, zeros_h)
    g2 = _stage2(s1, g1, dis, b1.reshape(1, H), W2)
    s2 = _sc_edge_sum(g2, src3, dst3, zeros_h)
    y = _stage3(s2, g2, dis, b2.reshape(1, H), Wo, bo.reshape(1, 1))
    return y[:N, 0]
